# baseline (speedup 1.0000x reference)
import functools

import jax
import jax.numpy as jnp
from jax import lax
from jax.experimental import pallas as pl
from jax.experimental.pallas import tpu as pltpu
from jax.experimental.pallas import tpu_sc as plsc

N = 10000
E = 320000
D = 128

NC = 2
NS = 16
LANES = 16

NTILE = NC * NS
EPT = E // NTILE
CHUNK = 80
NCHUNK = EPT // CHUNK
NPAD = 10240
VPS = NPAD // NS


def _sc_segment_sum(edge_features, receivers):
    mesh = plsc.VectorSubcoreMesh(core_axis_name="c", subcore_axis_name="s")

    @functools.partial(
        pl.kernel,
        out_type=jax.ShapeDtypeStruct((NC, NPAD, D), jnp.float32),
        mesh=mesh,
        scratch_types=[
            pltpu.VMEM((EPT,), jnp.int32),
            pltpu.VMEM((CHUNK, D), jnp.float32),
            pltpu.VMEM((CHUNK, D), jnp.float32),
            pltpu.VMEM((CHUNK, D), jnp.float32),
            pltpu.VMEM_SHARED((NPAD, D), jnp.float32),
            pltpu.SemaphoreType.DMA,
            pltpu.SemaphoreType.DMA,
            pltpu.SemaphoreType.DMA,
            pltpu.SemaphoreType.DMA,
            pltpu.SemaphoreType.DMA,
            pltpu.SemaphoreType.DMA,
            pltpu.SemaphoreType.DMA,
        ],
    )
    def k(ef_hbm, ei_hbm, out_hbm, idx_v, b0, b1, b2, accum,
          l0, l1, l2, s0, s1, s2, sem_i):
        c = lax.axis_index("c")
        s = lax.axis_index("s")
        bid = c * NS + s
        base = bid * EPT

        def _ld(x, buf, lsem):
            pltpu.async_copy(ef_hbm.at[pl.ds(base + x * CHUNK, CHUNK)], buf, lsem)

        def _lwait(buf, lsem):
            pltpu.make_async_copy(ef_hbm.at[pl.ds(0, CHUNK)], buf, lsem).wait()

        def _sc(x, buf, ssem):
            pltpu.async_copy(
                buf, accum.at[idx_v.at[pl.ds(x * CHUNK, CHUNK)]], ssem, add=True
            )

        pltpu.async_copy(ei_hbm.at[pl.ds(base, EPT)], idx_v, sem_i)
        _ld(0, b1, l1)
        _ld(1, b2, l2)

        @pl.loop(0, CHUNK)
        def _(r):
            @pl.loop(0, D, step=LANES)
            def _(l):
                b0[r, pl.ds(l, LANES)] = jnp.zeros((LANES,), jnp.float32)

        @pl.loop(0, VPS, step=CHUNK)
        def _(r0):
            pltpu.sync_copy(b0, accum.at[pl.ds(s * VPS + r0, CHUNK)])

        pltpu.make_async_copy(ei_hbm.at[pl.ds(0, EPT)], idx_v, sem_i).wait()
        plsc.subcore_barrier()

        _ld(2, b0, l0)
        _lwait(b1, l1)
        _sc(0, b1, s1)

        @pl.loop(1, 121, step=3)
        def _(x):
            _lwait(b2, l2)
            _sc(x, b2, s2)
            _lwait(b1, s1)
            _ld(x + 2, b1, l1)

            _lwait(b0, l0)
            _sc(x + 1, b0, s0)
            _lwait(b2, s2)
            _ld(x + 3, b2, l2)

            _lwait(b1, l1)
            _sc(x + 2, b1, s1)
            _lwait(b0, s0)
            _ld(x + 4, b0, l0)

        _lwait(b2, l2)
        _sc(121, b2, s2)
        _lwait(b1, s1)
        _ld(123, b1, l1)

        _lwait(b0, l0)
        _sc(122, b0, s0)
        _lwait(b2, s2)
        _ld(124, b2, l2)

        _lwait(b1, l1)
        _sc(123, b1, s1)
        _lwait(b0, s0)

        _lwait(b2, l2)
        _sc(124, b2, s2)
        _lwait(b1, s1)
        _lwait(b2, s2)

        plsc.subcore_barrier()

        pltpu.sync_copy(
            accum.at[pl.ds(s * VPS, VPS)], out_hbm.at[c].at[pl.ds(s * VPS, VPS)]
        )

    return k(edge_features, receivers)


BR = 2000


def _mm1_block(vf_ref, w1a_ref, b1_ref, h1_ref):
    h1_ref[...] = (
        jnp.dot(vf_ref[...], w1a_ref[...], preferred_element_type=jnp.float32)
        + b1_ref[...]
    )


def _tc_mm1(vertex_features, W1, b1):
    return pl.pallas_call(
        _mm1_block,
        grid=(N // BR,),
        in_specs=[
            pl.BlockSpec((BR, D), lambda i: (i, 0)),
            pl.BlockSpec((D, D), lambda i: (0, 0)),
            pl.BlockSpec((1, D), lambda i: (0, 0)),
        ],
        out_specs=pl.BlockSpec((BR, D), lambda i: (i, 0)),
        out_shape=jax.ShapeDtypeStruct((N, D), jnp.float32),
    )(vertex_features, W1[:D], b1.reshape(1, D))


def _mlp_block(h1_ref, p_ref, w1b_ref, w2_ref, b2_ref, o_ref):
    aggr = p_ref[0] + p_ref[1]
    h = h1_ref[...] + jnp.dot(
        aggr, w1b_ref[...], preferred_element_type=jnp.float32
    )
    h = jnp.maximum(h, 0.0)
    o_ref[...] = (
        jnp.dot(h, w2_ref[...], preferred_element_type=jnp.float32) + b2_ref[...]
    )


def _tc_mlp(h1, partials, W1, W2, b2):
    return pl.pallas_call(
        _mlp_block,
        grid=(N // BR,),
        in_specs=[
            pl.BlockSpec((BR, D), lambda i: (i, 0)),
            pl.BlockSpec((NC, BR, D), lambda i: (0, i, 0)),
            pl.BlockSpec((D, D), lambda i: (0, 0)),
            pl.BlockSpec((D, D), lambda i: (0, 0)),
            pl.BlockSpec((1, D), lambda i: (0, 0)),
        ],
        out_specs=pl.BlockSpec((BR, D), lambda i: (i, 0)),
        out_shape=jax.ShapeDtypeStruct((N, D), jnp.float32),
    )(h1, partials, W1[D:], W2, b2.reshape(1, D))


@jax.jit
def kernel(vertex_features, edge_features, edge_index, W1, b1, W2, b2):
    receivers = edge_index[1]
    h1 = _tc_mm1(vertex_features, W1, b1)
    partials = _sc_segment_sum(edge_features, receivers)
    return _tc_mlp(h1, partials, W1, W2, b2)


# speedup vs baseline: 1.0333x; 1.0333x over previous
import functools

import jax
import jax.numpy as jnp
from jax import lax
from jax.experimental import pallas as pl
from jax.experimental.pallas import tpu as pltpu
from jax.experimental.pallas import tpu_sc as plsc

N = 10000
E = 320000
D = 128

NC = 2
NS = 16
LANES = 16

NTILE = NC * NS
EPT = E // NTILE
CHUNK = 88
NCHUNK = 114
MAIN = (NCHUNK - 1) * CHUNK
TAIL_OFF = EPT - CHUNK
TAIL_DUP = CHUNK - (EPT - MAIN)
NPAD = 10240
TRASH = N
VPS = NPAD // NS
ZR = 80


def _sc_segment_sum(edge_features, ei_flat):
    mesh = plsc.VectorSubcoreMesh(core_axis_name="c", subcore_axis_name="s")

    @functools.partial(
        pl.kernel,
        out_type=jax.ShapeDtypeStruct((NC, NPAD, D), jnp.float32),
        mesh=mesh,
        scratch_types=[
            pltpu.VMEM((NCHUNK * CHUNK,), jnp.int32),
            pltpu.VMEM((CHUNK, D), jnp.float32),
            pltpu.VMEM((CHUNK, D), jnp.float32),
            pltpu.VMEM((CHUNK, D), jnp.float32),
            pltpu.VMEM_SHARED((NPAD, D), jnp.float32),
            pltpu.SemaphoreType.DMA,
            pltpu.SemaphoreType.DMA,
            pltpu.SemaphoreType.DMA,
            pltpu.SemaphoreType.DMA,
            pltpu.SemaphoreType.DMA,
            pltpu.SemaphoreType.DMA,
            pltpu.SemaphoreType.DMA,
        ],
    )
    def k(ef_hbm, ei_hbm, out_hbm, idx_v, b0, b1, b2, accum,
          l0, l1, l2, s0, s1, s2, sem_i):
        c = lax.axis_index("c")
        s = lax.axis_index("s")
        bid = c * NS + s
        base = bid * EPT

        def _ld(x, buf, lsem):
            off = base + jnp.minimum(x * CHUNK, TAIL_OFF)
            pltpu.async_copy(ef_hbm.at[pl.ds(off, CHUNK)], buf, lsem)

        def _lwait(buf, lsem):
            pltpu.make_async_copy(ef_hbm.at[pl.ds(0, CHUNK)], buf, lsem).wait()

        def _sc(x, buf, ssem):
            pltpu.async_copy(
                buf, accum.at[idx_v.at[pl.ds(x * CHUNK, CHUNK)]], ssem, add=True
            )

        pltpu.async_copy(
            ei_hbm.at[pl.ds(E + base, MAIN)], idx_v.at[pl.ds(0, MAIN)], sem_i
        )
        pltpu.async_copy(
            ei_hbm.at[pl.ds(E + base + MAIN, EPT - MAIN)],
            idx_v.at[pl.ds(MAIN + TAIL_DUP, EPT - MAIN)],
            sem_i,
        )
        _ld(0, b1, l1)
        _ld(1, b2, l2)

        @pl.loop(MAIN, MAIN + TAIL_DUP, step=LANES)
        def _(i):
            idx_v[pl.ds(i, LANES)] = jnp.full((LANES,), TRASH, jnp.int32)

        @pl.loop(0, ZR)
        def _(r):
            @pl.loop(0, D, step=LANES)
            def _(l):
                b0[r, pl.ds(l, LANES)] = jnp.zeros((LANES,), jnp.float32)

        @pl.loop(0, VPS, step=ZR)
        def _(r0):
            pltpu.sync_copy(
                b0.at[pl.ds(0, ZR)], accum.at[pl.ds(s * VPS + r0, ZR)]
            )

        pltpu.make_async_copy(
            ei_hbm.at[pl.ds(0, MAIN)], idx_v.at[pl.ds(0, MAIN)], sem_i
        ).wait()
        pltpu.make_async_copy(
            ei_hbm.at[pl.ds(0, EPT - MAIN)],
            idx_v.at[pl.ds(0, EPT - MAIN)],
            sem_i,
        ).wait()
        plsc.subcore_barrier()

        _ld(2, b0, l0)
        _lwait(b1, l1)
        _sc(0, b1, s1)

        @pl.loop(1, 112, step=3)
        def _(x):
            _lwait(b2, l2)
            _sc(x, b2, s2)
            _lwait(b1, s1)
            _ld(x + 2, b1, l1)

            _lwait(b0, l0)
            _sc(x + 1, b0, s0)
            _lwait(b2, s2)
            _ld(x + 3, b2, l2)

            _lwait(b1, l1)
            _sc(x + 2, b1, s1)
            _lwait(b0, s0)
            _ld(x + 4, b0, l0)

        _lwait(b2, l2)
        _sc(112, b2, s2)
        _lwait(b1, s1)

        _lwait(b0, l0)
        _sc(113, b0, s0)
        _lwait(b2, s2)
        _lwait(b0, s0)

        plsc.subcore_barrier()

        pltpu.sync_copy(
            accum.at[pl.ds(s * VPS, VPS)], out_hbm.at[c].at[pl.ds(s * VPS, VPS)]
        )

    return k(edge_features, ei_flat)


BR = 2000


def _mm1_block(vf_ref, w1a_ref, b1_ref, h1_ref):
    h1_ref[...] = (
        jnp.dot(vf_ref[...], w1a_ref[...], preferred_element_type=jnp.float32)
        + b1_ref[...]
    )


def _tc_mm1(vertex_features, W1, b1):
    return pl.pallas_call(
        _mm1_block,
        grid=(N // BR,),
        in_specs=[
            pl.BlockSpec((BR, D), lambda i: (i, 0)),
            pl.BlockSpec((D, D), lambda i: (0, 0)),
            pl.BlockSpec((1, D), lambda i: (0, 0)),
        ],
        out_specs=pl.BlockSpec((BR, D), lambda i: (i, 0)),
        out_shape=jax.ShapeDtypeStruct((N, D), jnp.float32),
    )(vertex_features, W1[:D], b1.reshape(1, D))


def _mlp_block(h1_ref, p_ref, w1b_ref, w2_ref, b2_ref, o_ref):
    aggr = p_ref[0] + p_ref[1]
    h = h1_ref[...] + jnp.dot(
        aggr, w1b_ref[...], preferred_element_type=jnp.float32
    )
    h = jnp.maximum(h, 0.0)
    o_ref[...] = (
        jnp.dot(h, w2_ref[...], preferred_element_type=jnp.float32) + b2_ref[...]
    )


def _tc_mlp(h1, partials, W1, W2, b2):
    return pl.pallas_call(
        _mlp_block,
        grid=(N // BR,),
        in_specs=[
            pl.BlockSpec((BR, D), lambda i: (i, 0)),
            pl.BlockSpec((NC, BR, D), lambda i: (0, i, 0)),
            pl.BlockSpec((D, D), lambda i: (0, 0)),
            pl.BlockSpec((D, D), lambda i: (0, 0)),
            pl.BlockSpec((1, D), lambda i: (0, 0)),
        ],
        out_specs=pl.BlockSpec((BR, D), lambda i: (i, 0)),
        out_shape=jax.ShapeDtypeStruct((N, D), jnp.float32),
    )(h1, partials, W1[D:], W2, b2.reshape(1, D))


@jax.jit
def kernel(vertex_features, edge_features, edge_index, W1, b1, W2, b2):
    ei_flat = edge_index.astype(jnp.int32).reshape(2 * E)
    h1 = _tc_mm1(vertex_features, W1, b1)
    partials = _sc_segment_sum(edge_features, ei_flat)
    return _tc_mlp(h1, partials, W1, W2, b2)


# speedup vs baseline: 1.0927x; 1.0574x over previous
import functools

import jax
import jax.numpy as jnp
from jax import lax
from jax.experimental import pallas as pl
from jax.experimental.pallas import tpu as pltpu
from jax.experimental.pallas import tpu_sc as plsc

N = 10000
E = 320000
D = 128

NC = 2
NS = 16
LANES = 16

NTILE = NC * NS
EPT = E // NTILE
CHUNK = 80
NCHUNK = EPT // CHUNK
NPAD = 10240
VPS = NPAD // NS


def _sc_segment_sum(edge_features, ei_flat):
    mesh = plsc.VectorSubcoreMesh(core_axis_name="c", subcore_axis_name="s")

    @functools.partial(
        pl.kernel,
        out_type=jax.ShapeDtypeStruct((NC, NPAD, D), jnp.float32),
        mesh=mesh,
        scratch_types=[
            pltpu.VMEM((EPT,), jnp.int32),
            pltpu.VMEM((CHUNK, D), jnp.float32),
            pltpu.VMEM((CHUNK, D), jnp.float32),
            pltpu.VMEM((CHUNK, D), jnp.float32),
            pltpu.VMEM_SHARED((NPAD, D), jnp.float32),
            pltpu.SemaphoreType.DMA,
            pltpu.SemaphoreType.DMA,
            pltpu.SemaphoreType.DMA,
            pltpu.SemaphoreType.DMA,
            pltpu.SemaphoreType.DMA,
            pltpu.SemaphoreType.DMA,
            pltpu.SemaphoreType.DMA,
        ],
    )
    def k(ef_hbm, ei_hbm, out_hbm, idx_v, b0, b1, b2, accum,
          l0, l1, l2, s0, s1, s2, sem_i):
        c = lax.axis_index("c")
        s = lax.axis_index("s")
        bid = c * NS + s
        base = bid * EPT

        def _ld(x, buf, lsem):
            pltpu.async_copy(ef_hbm.at[pl.ds(base + x * CHUNK, CHUNK)], buf, lsem)

        def _lwait(buf, lsem):
            pltpu.make_async_copy(ef_hbm.at[pl.ds(0, CHUNK)], buf, lsem).wait()

        def _sc(x, buf, ssem):
            pltpu.async_copy(
                buf, accum.at[idx_v.at[pl.ds(x * CHUNK, CHUNK)]], ssem, add=True
            )

        pltpu.async_copy(ei_hbm.at[pl.ds(E + base, EPT)], idx_v, sem_i)
        _ld(0, b1, l1)
        _ld(1, b2, l2)

        @pl.loop(0, CHUNK)
        def _(r):
            @pl.loop(0, D, step=LANES)
            def _(l):
                b0[r, pl.ds(l, LANES)] = jnp.zeros((LANES,), jnp.float32)

        @pl.loop(0, VPS, step=CHUNK)
        def _(r0):
            pltpu.sync_copy(b0, accum.at[pl.ds(s * VPS + r0, CHUNK)])

        pltpu.make_async_copy(ei_hbm.at[pl.ds(0, EPT)], idx_v, sem_i).wait()
        plsc.subcore_barrier()

        _ld(2, b0, l0)
        _lwait(b1, l1)
        _sc(0, b1, s1)

        @pl.loop(1, 121, step=3)
        def _(x):
            _lwait(b2, l2)
            _sc(x, b2, s2)
            _lwait(b1, s1)
            _ld(x + 2, b1, l1)

            _lwait(b0, l0)
            _sc(x + 1, b0, s0)
            _lwait(b2, s2)
            _ld(x + 3, b2, l2)

            _lwait(b1, l1)
            _sc(x + 2, b1, s1)
            _lwait(b0, s0)
            _ld(x + 4, b0, l0)

        _lwait(b2, l2)
        _sc(121, b2, s2)
        _lwait(b1, s1)
        _ld(123, b1, l1)

        _lwait(b0, l0)
        _sc(122, b0, s0)
        _lwait(b2, s2)
        _ld(124, b2, l2)

        _lwait(b1, l1)
        _sc(123, b1, s1)
        _lwait(b0, s0)

        _lwait(b2, l2)
        _sc(124, b2, s2)
        _lwait(b1, s1)
        _lwait(b2, s2)

        plsc.subcore_barrier()

        pltpu.sync_copy(
            accum.at[pl.ds(s * VPS, VPS)], out_hbm.at[c].at[pl.ds(s * VPS, VPS)]
        )

    return k(edge_features, ei_flat)


BR = 2000


def _mm1_block(vf_ref, w1a_ref, b1_ref, h1_ref):
    h1_ref[...] = (
        jnp.dot(vf_ref[...], w1a_ref[...], preferred_element_type=jnp.float32)
        + b1_ref[...]
    ).astype(jnp.bfloat16)


def _tc_mm1(vertex_features, W1, b1):
    return pl.pallas_call(
        _mm1_block,
        grid=(N // BR,),
        in_specs=[
            pl.BlockSpec((BR, D), lambda i: (i, 0)),
            pl.BlockSpec((D, D), lambda i: (0, 0)),
            pl.BlockSpec((1, D), lambda i: (0, 0)),
        ],
        out_specs=pl.BlockSpec((BR, D), lambda i: (i, 0)),
        out_shape=jax.ShapeDtypeStruct((N, D), jnp.bfloat16),
    )(vertex_features, W1[:D], b1.reshape(1, D))


def _mlp_block(h1_ref, p_ref, w1b_ref, w2_ref, b2_ref, o_ref):
    aggr = p_ref[0] + p_ref[1]
    h = h1_ref[...].astype(jnp.float32) + jnp.dot(
        aggr, w1b_ref[...], preferred_element_type=jnp.float32
    )
    h = jnp.maximum(h, 0.0)
    o_ref[...] = (
        jnp.dot(h, w2_ref[...], preferred_element_type=jnp.float32) + b2_ref[...]
    )


def _tc_mlp(h1, partials, W1, W2, b2):
    return pl.pallas_call(
        _mlp_block,
        grid=(N // BR,),
        in_specs=[
            pl.BlockSpec((BR, D), lambda i: (i, 0)),
            pl.BlockSpec((NC, BR, D), lambda i: (0, i, 0)),
            pl.BlockSpec((D, D), lambda i: (0, 0)),
            pl.BlockSpec((D, D), lambda i: (0, 0)),
            pl.BlockSpec((1, D), lambda i: (0, 0)),
        ],
        out_specs=pl.BlockSpec((BR, D), lambda i: (i, 0)),
        out_shape=jax.ShapeDtypeStruct((N, D), jnp.float32),
    )(h1, partials, W1[D:], W2, b2.reshape(1, D))


@jax.jit
def kernel(vertex_features, edge_features, edge_index, W1, b1, W2, b2):
    ei_flat = edge_index.astype(jnp.int32).reshape(2 * E)
    h1 = _tc_mm1(vertex_features, W1, b1)
    partials = _sc_segment_sum(edge_features, ei_flat)
    return _tc_mlp(h1, partials, W1, W2, b2)
